# Initial kernel scaffold; baseline (speedup 1.0000x reference)
import jax
import jax.numpy as jnp
from jax.experimental import pallas as pl


def kernel(pos, batch, params):
    raise NotImplementedError("write your pallas kernel here")


# speedup vs baseline: 5.8923x; 5.8923x over previous
import functools

import jax
import jax.numpy as jnp
import numpy as np
from jax.experimental import pallas as pl

_B = 16
_P0 = 2048
_DEC = 4
_K = 16
_LEVELS = [(3, 32), (16, 128), (64, 256), (128, 512)]
_NEG_SLOPE = 0.01


def _dot(a, b):
    return jnp.dot(a, b, preferred_element_type=jnp.float32)


def _lfa_layer(xT, pos, posq, posT, posqT, w, din, dout):
    h = dout // 2
    P = pos.shape[0]
    Pq = posq.shape[0]

    hT = _dot(w['W_m1T'], xT) + w['b_m1']
    featT = jnp.concatenate([hT, posT], axis=0)

    ip = jax.lax.broadcasted_iota(jnp.int32, (P, Pq), 0)
    iq = jax.lax.broadcasted_iota(jnp.int32, (P, Pq), 1)
    dmatT = (ip == _DEC * iq).astype(jnp.float32)
    xqT = _dot(xT, dmatT)
    scqT = _dot(w['W_scT'], xqT) + w['b_sc']

    dx = pos[:, 0:1] - posqT[0:1, :]
    dy = pos[:, 1:2] - posqT[1:2, :]
    dz = pos[:, 2:3] - posqT[2:3, :]
    d2T = (dx * dx + dy * dy) + dz * dz

    ip = jax.lax.broadcasted_iota(jnp.int32, (P, Pq), 0)
    taken = jnp.zeros((P, Pq), jnp.bool_)
    Tacc = jnp.zeros((1, Pq), jnp.float32)
    for _ in range(_K):
        d2m = jnp.where(taken, jnp.inf, d2T)
        t = jnp.min(d2m, axis=0, keepdims=True)
        sel = jnp.where(d2m == t, ip, P)
        j = jnp.min(sel, axis=0, keepdims=True)
        oneb = sel == j
        taken = taken | oneb
        oneh = oneb.astype(jnp.float32)
        gT = _dot(featT, oneh)
        xjT = gT[:h]
        dT = gT[h:h + 3] - posqT
        encT = _dot(w['W_l2aT'], dT) + _dot(w['W_l2bT'], dT * dT) + w['b_l2']
        sT = _dot(w['W_p2aT'], xjT) + _dot(w['W_p2bT'], encT) + w['b_p2']
        sT = sT - jnp.max(sT, axis=0, keepdims=True)
        e = jnp.exp(sT)
        aT = e / jnp.sum(e, axis=0, keepdims=True)
        Tacc = Tacc + (jnp.sum(xjT * aT[:h], axis=0, keepdims=True)
                       + jnp.sum(encT * aT[h:], axis=0, keepdims=True))

    aggT = w['W_m2T'] * Tacc + _K * w['b_m2']
    pre = aggT + scqT
    return jnp.where(pre >= 0, pre, _NEG_SLOPE * pre)


def _mega_body(*refs):
    pos_refs = refs[0:5]
    posT_refs = refs[5:10]
    wflat = refs[10:10 + 4 * 12]
    W_gpT_ref, b_gp_ref = refs[58], refs[59]
    out_ref = refs[60]

    xT = posT_refs[0][0]
    for l, (din, dout) in enumerate(_LEVELS):
        w = {}
        keys = ['W_m1T', 'b_m1', 'W_scT', 'b_sc', 'W_l2aT', 'W_l2bT',
                'b_l2', 'W_p2aT', 'W_p2bT', 'b_p2', 'W_m2T', 'b_m2']
        for j, k in enumerate(keys):
            w[k] = wflat[l * 12 + j][...]
        xT = _lfa_layer(xT, pos_refs[l][0], pos_refs[l + 1][0],
                        posT_refs[l][0], posT_refs[l + 1][0], w, din, dout)

    gcat = jnp.concatenate([xT, posT_refs[4][0]], axis=0)
    gT = _dot(W_gpT_ref[...], gcat) + b_gp_ref[...]
    out_ref[0] = jnp.max(gT, axis=1, keepdims=True)


def _head_body(pooled_ref, w1, b1, w2, b2, w3, b3, out_ref):
    h1 = jnp.maximum(_dot(pooled_ref[...], w1[...]) + b1[...], 0.0)
    h2 = jnp.maximum(_dot(h1, w2[...]) + b2[...], 0.0)
    logits = _dot(h2, w3[...]) + b3[...]
    m = jnp.max(logits, axis=-1, keepdims=True)
    s = logits - m
    lse = jnp.log(jnp.sum(jnp.exp(s), axis=-1, keepdims=True))
    out_ref[...] = s - lse


def kernel(pos, batch, params):
    del batch
    f32 = jnp.float32
    pos3 = pos.reshape(_B, _P0, 3)

    pos_lv = []
    posT_lv = []
    p = pos3
    for _ in range(5):
        pos_lv.append(p)
        posT_lv.append(jnp.transpose(p, (0, 2, 1)))
        p = p[:, ::_DEC]

    wflat = []
    for l, (din, dout) in enumerate(_LEVELS):
        h = dout // 2
        lp = params['l%d' % (l + 1)]
        wflat += [
            lp['W_m1'].T, lp['b_m1'].reshape(h, 1),
            lp['W_sc'].T, lp['b_sc'].reshape(h, 1),
            lp['W_l2'].T[:, 0:3], lp['W_l2'].T[:, 3:6],
            lp['b_l2'].reshape(h, 1),
            lp['W_p2'].T[:, 0:h], lp['W_p2'].T[:, h:dout],
            lp['b_p2'].reshape(dout, 1),
            lp['W_m2'].T, lp['b_m2'].reshape(h, 1),
        ]
    W_gpT = params['W_gp'].T
    b_gp = params['b_gp'].reshape(1024, 1)

    full = lambda a: pl.BlockSpec(a.shape, lambda i: (0,) * a.ndim)
    cloud = lambda a: pl.BlockSpec((1,) + a.shape[1:],
                                   lambda i: (i,) + (0,) * (a.ndim - 1))
    operands = pos_lv + posT_lv + wflat + [W_gpT, b_gp]
    in_specs = ([cloud(a) for a in pos_lv + posT_lv]
                + [full(a) for a in wflat + [W_gpT, b_gp]])

    pooled = pl.pallas_call(
        _mega_body,
        grid=(_B,),
        in_specs=in_specs,
        out_specs=pl.BlockSpec((1, 1024, 1), lambda i: (i, 0, 0)),
        out_shape=jax.ShapeDtypeStruct((_B, 1024, 1), f32),
    )(*operands)

    pooled2 = pooled.reshape(_B, 1024)
    head_ops = [pooled2,
                params['W_f1'], params['b_f1'].reshape(1, 512),
                params['W_f2'], params['b_f2'].reshape(1, 256),
                params['W_f3'], params['b_f3'].reshape(1, 10)]
    out = pl.pallas_call(
        _head_body,
        in_specs=[pl.BlockSpec(a.shape, lambda: (0,) * a.ndim)
                  for a in head_ops],
        out_specs=pl.BlockSpec((_B, 10), lambda: (0, 0)),
        out_shape=jax.ShapeDtypeStruct((_B, 10), f32),
    )(*head_ops)
    return out
